# baseline (speedup 1.0000x reference)
import functools

import jax
import jax.numpy as jnp
from jax import lax
from jax.experimental import pallas as pl
from jax.experimental.pallas import tpu as pltpu
from jax.experimental.pallas import tpu_sc as plsc

V = 1000000
D = 64
DW = 128
B = 16384 * 50
NC, NS = 2, 16
NW = NC * NS

ACH = 400
ANCH = V // ACH
AITER = (ANCH + NW - 1) // NW

BPW = B // NW
CH = 256
IPG = 128
NG = CH // IPG
NCHUNK = BPW // CH


@functools.lru_cache(maxsize=1)
def _build():
    mesh = plsc.VectorSubcoreMesh(core_axis_name="c", subcore_axis_name="s")

    @functools.partial(
        pl.kernel,
        mesh=mesh,
        out_type=jax.ShapeDtypeStruct((V, DW), jnp.float32),
        scratch_types=[
            pltpu.VMEM((ACH, D), jnp.float32),
            pltpu.VMEM((ACH, DW), jnp.float32),
        ],
    )
    def depad(mat_hbm, wide_hbm, m_v, b_v):
        wid = lax.axis_index("s") * NC + lax.axis_index("c")

        def step(i, carry):
            c = wid + i * NW

            @pl.when(c < ANCH)
            def _():
                base = c * ACH
                pltpu.sync_copy(mat_hbm.at[pl.ds(base, ACH)], m_v)

                def rows(r, carry2):
                    for u in range(8):
                        for k in range(4):
                            sl = pl.ds(k * 16, 16)
                            b_v[r * 8 + u, sl] = m_v[r * 8 + u, sl]
                    return carry2

                lax.fori_loop(0, ACH // 8, rows, 0)
                pltpu.sync_copy(b_v, wide_hbm.at[pl.ds(base, ACH)])

            return carry

        lax.fori_loop(0, AITER, step, 0)

    @functools.partial(
        pl.kernel,
        mesh=mesh,
        out_type=jax.ShapeDtypeStruct((B, D), jnp.float32),
        scratch_types=[
            pltpu.VMEM((BPW,), jnp.int32),
            pltpu.VMEM((CH, DW), jnp.float32),
            pltpu.VMEM((CH, D), jnp.float32),
            pltpu.SemaphoreType.DMA,
        ],
    )
    def gather(wide_hbm, idx_hbm, out_hbm, idx_v, rows_v, o_v, sem):
        wid = lax.axis_index("s") * NC + lax.axis_index("c")
        wbase = wid * BPW
        pltpu.sync_copy(idx_hbm.at[pl.ds(wbase, BPW)], idx_v)

        def chunk(g, carry):
            cbase = g * CH
            copies = []
            for j in range(NG):
                copies.append(pltpu.async_copy(
                    wide_hbm.at[idx_v.at[pl.ds(cbase + j * IPG, IPG)]],
                    rows_v.at[pl.ds(j * IPG, IPG)],
                    sem))
            for c in copies:
                c.wait()

            def rows(r, carry2):
                for u in range(8):
                    for k in range(4):
                        sl = pl.ds(k * 16, 16)
                        o_v[r * 8 + u, sl] = rows_v[r * 8 + u, sl]
                return carry2

            lax.fori_loop(0, CH // 8, rows, 0)
            pltpu.sync_copy(o_v, out_hbm.at[pl.ds(wbase + cbase, CH)])
            return carry

        lax.fori_loop(0, NCHUNK, chunk, 0)

    def run(mat, idx):
        wide = depad(mat)
        return gather(wide, idx)

    return run


def kernel(x, mat):
    idx = x.reshape(-1).astype(jnp.int32)
    return _build()(mat, idx)


# speedup vs baseline: 1.4563x; 1.4563x over previous
import functools

import jax
import jax.numpy as jnp
from jax import lax
from jax.experimental import pallas as pl
from jax.experimental.pallas import tpu as pltpu
from jax.experimental.pallas import tpu_sc as plsc

V = 1000000
D = 64
DW = 128
B = 16384 * 50
NC, NS = 2, 16
NW = NC * NS
BPW = B // NW
CH = 512
IPG = 128
NG = CH // IPG
NCHUNK = BPW // CH


@functools.lru_cache(maxsize=1)
def _build():
    mesh = plsc.VectorSubcoreMesh(core_axis_name="c", subcore_axis_name="s")

    @functools.partial(
        pl.kernel,
        mesh=mesh,
        out_type=jax.ShapeDtypeStruct((B, DW), jnp.float32),
        scratch_types=[
            pltpu.VMEM((BPW,), jnp.int32),
            pltpu.VMEM((CH, DW), jnp.float32),
            pltpu.SemaphoreType.DMA,
        ],
    )
    def gather(wide_hbm, idx_hbm, out_hbm, idx_v, rows_v, sem):
        wid = lax.axis_index("s") * NC + lax.axis_index("c")
        wbase = wid * BPW
        pltpu.sync_copy(idx_hbm.at[pl.ds(wbase, BPW)], idx_v)

        def chunk(g, carry):
            cbase = g * CH
            copies = []
            for j in range(NG):
                copies.append(pltpu.async_copy(
                    wide_hbm.at[idx_v.at[pl.ds(cbase + j * IPG, IPG)]],
                    rows_v.at[pl.ds(j * IPG, IPG)],
                    sem))
            for c in copies:
                c.wait()
            pltpu.sync_copy(rows_v, out_hbm.at[pl.ds(wbase + cbase, CH)])
            return carry

        lax.fori_loop(0, NCHUNK, chunk, 0)

    return gather


def kernel(x, mat):
    idx = x.reshape(-1).astype(jnp.int32)
    wide = jnp.pad(mat, ((0, 0), (0, DW - D)))
    out_w = _build()(wide, idx)
    return out_w[:, :D]


# speedup vs baseline: 1.4855x; 1.0200x over previous
import functools

import jax
import jax.numpy as jnp
from jax import lax
from jax.experimental import pallas as pl
from jax.experimental.pallas import tpu as pltpu
from jax.experimental.pallas import tpu_sc as plsc

V = 1000000
D = 64
DW = 128
B = 16384 * 50
NC, NS = 2, 16
NW = NC * NS
BPW = B // NW
CH = 128
NCHUNK = BPW // CH
NB = 5
NMAIN = NCHUNK - NB


@functools.lru_cache(maxsize=1)
def _build():
    mesh = plsc.VectorSubcoreMesh(core_axis_name="c", subcore_axis_name="s")

    @functools.partial(
        pl.kernel,
        mesh=mesh,
        out_type=jax.ShapeDtypeStruct((B, DW), jnp.float32),
        scratch_types=(
            [pltpu.VMEM((BPW,), jnp.int32)]
            + [pltpu.VMEM((CH, DW), jnp.float32) for _ in range(NB)]
            + [pltpu.SemaphoreType.DMA for _ in range(2 * NB)]
        ),
    )
    def gather(wide_hbm, idx_hbm, out_hbm, idx_v, *bufs_and_sems):
        rows = bufs_and_sems[:NB]
        gsem = bufs_and_sems[NB:2 * NB]
        osem = bufs_and_sems[2 * NB:]

        wid = lax.axis_index("s") * NC + lax.axis_index("c")
        wbase = wid * BPW
        pltpu.sync_copy(idx_hbm.at[pl.ds(wbase, BPW)], idx_v)

        def start_gather(g, b):
            pltpu.async_copy(
                wide_hbm.at[idx_v.at[pl.ds(g * CH, CH)]], rows[b], gsem[b])

        def wait_gather(b):
            pltpu.make_async_copy(
                wide_hbm.at[pl.ds(0, CH)], rows[b], gsem[b]).wait()

        def start_out(g, b):
            pltpu.async_copy(
                rows[b], out_hbm.at[pl.ds(wbase + g * CH, CH)], osem[b])

        def wait_out(g, b):
            pltpu.make_async_copy(
                rows[b], out_hbm.at[pl.ds(wbase + g * CH, CH)], osem[b]).wait()

        for b in range(NB):
            start_gather(b, b)

        def main(i, carry):
            g0 = i * NB
            for b in range(NB):
                g = g0 + b
                wait_gather(b)
                start_out(g, b)
                wait_out(g, b)
                start_gather(g + NB, b)
            return carry

        lax.fori_loop(0, NMAIN // NB, main, 0)

        for b in range(NB):
            g = NMAIN + b
            wait_gather(b)
            pltpu.sync_copy(rows[b], out_hbm.at[pl.ds(wbase + g * CH, CH)])

    return gather


def kernel(x, mat):
    idx = x.reshape(-1).astype(jnp.int32)
    wide = jnp.pad(mat, ((0, 0), (0, DW - D)))
    out_w = _build()(wide, idx)
    return out_w[:, :D]
